# baseline (speedup 1.0000x reference)
import functools
import math

import jax
import jax.numpy as jnp
from jax import lax
from jax.experimental import pallas as pl
from jax.experimental.pallas import tpu as pltpu
from jax.experimental.pallas import tpu_sc as plsc

_EPS = 1e-5
_K = 8
_CH = [(3, 32), (32, 64), (64, 128), (128, 256), (256, 512)]
_BNS = float(1.0 / math.sqrt(1.0 + _EPS))

_SC_NC = 2
_SC_NS = 16
_SC_NW = _SC_NC * _SC_NS
_SC_CHUNK = 128


def _bf(a):
    return a.astype(jnp.bfloat16).astype(jnp.float32)


def _softmax_rows(a):
    m = jnp.max(a, axis=1, keepdims=True)
    e = jnp.exp(a - m)
    return e / jnp.sum(e, axis=1, keepdims=True)


def _topk_dist(q, xb, qn, xn):
    R = q.shape[0]
    N = xb.shape[0]
    d = -2.0 * lax.dot_general(q, xb, (((1,), (1,)), ((), ())),
                               preferred_element_type=jnp.float32)
    d = d + qn
    d = d + xn
    iota = lax.broadcasted_iota(jnp.int32, (R, N), 1)
    sels = []
    for _ in range(_K):
        m = jnp.min(d, axis=1, keepdims=True)
        sel = jnp.min(jnp.where(d == m, iota, N), axis=1, keepdims=True)
        sels.append(sel)
        d = jnp.where(iota == sel, jnp.float32(1e30), d)
    return sels


def _xform(xt, wxt_ref, bxt_ref, gxt_ref, bexr_ref):
    X = lax.dot_general(xt, wxt_ref[...], (((1,), (1,)), ((), ())),
                        preferred_element_type=jnp.float32)
    X = X + bxt_ref[...][None, :]
    X = X * _BNS * gxt_ref[...][None, :] + bexr_ref[...][None, :]
    return jnp.concatenate(
        [_softmax_rows(X[:, i * _K:(i + 1) * _K]) for i in range(_K)], axis=1)


def _out_proj(g, wc_ref, bc_ref, gc_ref, bec_ref):
    out = lax.dot_general(g, wc_ref[...], (((1,), (1,)), ((), ())),
                          preferred_element_type=jnp.float32)
    out = out + bc_ref[...][None, :]
    out = out * _BNS * gc_ref[...][None, :] + bec_ref[...][None, :]
    return jnp.maximum(out, 0.0)


def _layer0_kernel(xb_ref, q_ref, qn_ref, xn_ref, wxt_ref, bxt_ref, gxt_ref,
                   bexr_ref, wc_ref, bc_ref, gc_ref, bec_ref, out_ref):
    xb = xb_ref[0]
    q = q_ref[0]
    N = xb.shape[0]
    sels = _topk_dist(q, xb, qn_ref[0], xn_ref[0])
    iota = lax.broadcasted_iota(jnp.int32, sels[0].shape[:1] + (N,), 1)
    xg = [lax.dot_general((iota == s).astype(jnp.float32), xb,
                          (((1,), (0,)), ((), ())),
                          preferred_element_type=jnp.float32) for s in sels]
    s = _xform(jnp.concatenate([g[:, 0:1] for g in xg], axis=1),
               wxt_ref, bxt_ref, gxt_ref, bexr_ref)
    sb = _bf(s)
    xgb = [_bf(gj) for gj in xg]
    g = None
    for i in range(_K):
        si = sb[:, i * _K:(i + 1) * _K]
        terms = [si[:, j:j + 1] * xgb[j] for j in range(_K)]
        while len(terms) > 1:
            terms = [terms[k] + terms[k + 1] for k in range(0, len(terms), 2)]
        g = terms[0] if g is None else jnp.maximum(g, terms[0])
    out_ref[0] = _out_proj(g, wc_ref, bc_ref, gc_ref, bec_ref)


def _xconv_layer0(x, p, c_out, block_r=256):
    B, N, C = x.shape
    s2 = jnp.sum(x ** 2, axis=-1)
    full = lambda shape: pl.BlockSpec(shape, lambda b, r: (0,) * len(shape))
    return pl.pallas_call(
        _layer0_kernel,
        grid=(B, N // block_r),
        in_specs=[
            pl.BlockSpec((1, N, C), lambda b, r: (b, 0, 0)),
            pl.BlockSpec((1, block_r, C), lambda b, r: (b, r, 0)),
            pl.BlockSpec((1, block_r, 1), lambda b, r: (b, r, 0)),
            pl.BlockSpec((1, 1, N), lambda b, r: (b, 0, 0)),
            full((_K * _K, _K)), full((_K * _K,)), full((_K * _K,)),
            full((_K * _K,)),
            full((c_out, C)), full((c_out,)), full((c_out,)), full((c_out,)),
        ],
        out_specs=pl.BlockSpec((1, block_r, c_out), lambda b, r: (b, r, 0)),
        out_shape=jax.ShapeDtypeStruct((B, N, c_out), jnp.float32),
    )(x, x, s2[:, :, None], s2[:, None, :],
      p['Wxt'], p['bxt'], p['gxt'], p['betaxt'],
      p['Wc'], p['bc'], p['gc'], p['betac'])


def _knn_kernel(xb_ref, q_ref, qn_ref, xn_ref, idx_ref):
    xb = xb_ref[0]
    q = q_ref[0]
    N = xb.shape[0]
    b = pl.program_id(0)
    sels = _topk_dist(q, xb, qn_ref[0], xn_ref[0])
    idx_ref[0] = jnp.concatenate(sels, axis=1) + b * N


def _knn_stage(x, block_r=256):
    B, N, C = x.shape
    s2 = jnp.sum(x ** 2, axis=-1)
    return pl.pallas_call(
        _knn_kernel,
        grid=(B, N // block_r),
        in_specs=[
            pl.BlockSpec((1, N, C), lambda b, r: (b, 0, 0)),
            pl.BlockSpec((1, block_r, C), lambda b, r: (b, r, 0)),
            pl.BlockSpec((1, block_r, 1), lambda b, r: (b, r, 0)),
            pl.BlockSpec((1, 1, N), lambda b, r: (b, 0, 0)),
        ],
        out_specs=pl.BlockSpec((1, block_r, _K), lambda b, r: (b, r, 0)),
        out_shape=jax.ShapeDtypeStruct((B, N, _K), jnp.int32),
    )(x, x, s2[:, :, None], s2[:, None, :])


def _sc_gather(feat, idx3, c):
    nch = idx3.shape[1]
    total = _SC_NW * nch * _SC_CHUNK
    per_w = nch * _SC_CHUNK
    mesh = plsc.VectorSubcoreMesh(core_axis_name="c", subcore_axis_name="s")

    @functools.partial(
        pl.kernel,
        out_type=jax.ShapeDtypeStruct((total, c), jnp.float32),
        mesh=mesh,
        scratch_types=[
            pltpu.VMEM((nch, _SC_CHUNK), jnp.int32),
            pltpu.VMEM((_SC_CHUNK, c), jnp.float32),
            pltpu.SemaphoreType.DMA,
        ],
    )
    def gather_k(feat_hbm, idx_hbm, out_hbm, idx_v, rows_v, sem):
        wid = lax.axis_index("s") * _SC_NC + lax.axis_index("c")
        pltpu.sync_copy(idx_hbm.at[wid], idx_v)

        def body(t, carry):
            pltpu.async_copy(feat_hbm.at[idx_v.at[t]], rows_v, sem).wait()
            pltpu.sync_copy(
                rows_v,
                out_hbm.at[pl.ds(wid * per_w + t * _SC_CHUNK, _SC_CHUNK)])
            return carry

        lax.fori_loop(0, nch, body, 0)

    return gather_k(feat, idx3)


def _combine_kernel(g_ref, wxt_ref, bxt_ref, gxt_ref, bexr_ref,
                    wc_ref, bc_ref, gc_ref, bec_ref, out_ref):
    gav = [_bf(g_ref[:, j, :]) for j in range(_K)]
    xt = jnp.concatenate([g_ref[:, j, 0:1] for j in range(_K)], axis=1)
    s = _bf(_xform(xt, wxt_ref, bxt_ref, gxt_ref, bexr_ref))
    g = None
    for i in range(_K):
        si = s[:, i * _K:(i + 1) * _K]
        terms = [si[:, j:j + 1] * gav[j] for j in range(_K)]
        while len(terms) > 1:
            terms = [terms[k] + terms[k + 1] for k in range(0, len(terms), 2)]
        g = terms[0] if g is None else jnp.maximum(g, terms[0])
    out_ref[...] = _out_proj(g, wc_ref, bc_ref, gc_ref, bec_ref)


def _combine_stage(G, wc, p, c_out, block_r=512):
    M, _, C = G.shape
    full = lambda shape: pl.BlockSpec(shape, lambda r: (0,) * len(shape))
    return pl.pallas_call(
        _combine_kernel,
        grid=(M // block_r,),
        in_specs=[
            pl.BlockSpec((block_r, _K, C), lambda r: (r, 0, 0)),
            full((_K * _K, _K)), full((_K * _K,)), full((_K * _K,)),
            full((_K * _K,)),
            full((c_out, C)), full((c_out,)), full((c_out,)), full((c_out,)),
        ],
        out_specs=pl.BlockSpec((block_r, c_out), lambda r: (r, 0)),
        out_shape=jax.ShapeDtypeStruct((M, c_out), jnp.float32),
    )(G, p['Wxt'], p['bxt'], p['gxt'], p['betaxt'],
      wc, p['bc'], p['gc'], p['betac'])


def _head_kernel(h_ref, w1_ref, b1_ref, w2_ref, b2_ref, out_ref):
    h = h_ref[0]
    m = jnp.max(h, axis=0, keepdims=True)
    f = lax.dot_general(m, w1_ref[...], (((1,), (1,)), ((), ())),
                        preferred_element_type=jnp.float32)
    f = jnp.maximum(f + b1_ref[...][None, :], 0.0)
    o = lax.dot_general(f, w2_ref[...], (((1,), (1,)), ((), ())),
                        preferred_element_type=jnp.float32)
    out_ref[0, 0] = o[0] + b2_ref[...]


def _head(h, params):
    B, N, C = h.shape
    full = lambda shape: pl.BlockSpec(shape, lambda b: (0,) * len(shape))
    return pl.pallas_call(
        _head_kernel,
        grid=(B,),
        in_specs=[
            pl.BlockSpec((1, N, C), lambda b: (b, 0, 0)),
            full((256, 512)), full((256,)), full((40, 256)), full((40,)),
        ],
        out_specs=pl.BlockSpec((1, 1, 40), lambda b: (b, 0, 0)),
        out_shape=jax.ShapeDtypeStruct((B, 1, 40), jnp.float32),
    )(h, params['fc1_w'], params['fc1_b'], params['fc2_w'],
      params['fc2_b']).reshape(B, 40)


def _trunk(x, params):
    B, N, _ = x.shape
    h = _xconv_layer0(x, params['xconv0'], _CH[0][1])
    for i in range(1, len(_CH)):
        p = params['xconv%d' % i]
        C, c_out = _CH[i]
        idx = _knn_stage(h)
        idx3 = idx.reshape(_SC_NW, (B * N * _K) // (_SC_NW * _SC_CHUNK),
                           _SC_CHUNK)
        Cp = max(C, 128)
        feat = h.reshape(B * N, C)
        wc = p['Wc']
        if Cp != C:
            feat = jnp.pad(feat, ((0, 0), (0, Cp - C)))
            wc = jnp.pad(wc, ((0, 0), (0, Cp - C)))
        G = _sc_gather(feat, idx3, Cp)
        h = _combine_stage(G.reshape(B * N, _K, Cp), wc, p, c_out)
        h = h.reshape(B, N, c_out)
    return h


@jax.jit
def kernel(x, params):
    B = x.shape[0]
    Q = B // 4
    h = jnp.concatenate([_trunk(x[i * Q:(i + 1) * Q], params)
                         for i in range(4)], axis=0)
    return _head(h, params)


# speedup vs baseline: 1.1054x; 1.1054x over previous
import functools
import math

import jax
import jax.numpy as jnp
from jax import lax
from jax.experimental import pallas as pl
from jax.experimental.pallas import tpu as pltpu
from jax.experimental.pallas import tpu_sc as plsc

_EPS = 1e-5
_K = 8
_CH = [(3, 32), (32, 64), (64, 128), (128, 256), (256, 512)]
_BNS = float(1.0 / math.sqrt(1.0 + _EPS))

_SC_NC = 2
_SC_NS = 16
_SC_NW = _SC_NC * _SC_NS
_SC_CHUNK = 128


def _bf(a):
    return a.astype(jnp.bfloat16).astype(jnp.float32)


def _softmax_rows(a):
    m = jnp.max(a, axis=1, keepdims=True)
    e = jnp.exp(a - m)
    return e / jnp.sum(e, axis=1, keepdims=True)


def _topk_dist(q, xb, qn, xn):
    R = q.shape[0]
    N = xb.shape[0]
    d = -2.0 * lax.dot_general(q, xb, (((1,), (1,)), ((), ())),
                               preferred_element_type=jnp.float32)
    d = d + qn
    d = d + xn
    iota = lax.broadcasted_iota(jnp.int32, (R, N), 1)
    sels = []
    for _ in range(_K):
        m = jnp.min(d, axis=1, keepdims=True)
        sel = jnp.min(jnp.where(d == m, iota, N), axis=1, keepdims=True)
        sels.append(sel)
        d = jnp.where(iota == sel, jnp.float32(1e30), d)
    return sels


def _xform(xt, wxt_ref, bxt_ref, gxt_ref, bexr_ref):
    X = lax.dot_general(xt, wxt_ref[...], (((1,), (1,)), ((), ())),
                        preferred_element_type=jnp.float32)
    X = X + bxt_ref[...][None, :]
    X = X * _BNS * gxt_ref[...][None, :] + bexr_ref[...][None, :]
    return jnp.concatenate(
        [_softmax_rows(X[:, i * _K:(i + 1) * _K]) for i in range(_K)], axis=1)


def _out_proj(g, wc_ref, bc_ref, gc_ref, bec_ref):
    out = lax.dot_general(g, wc_ref[...], (((1,), (1,)), ((), ())),
                          preferred_element_type=jnp.float32)
    out = out + bc_ref[...][None, :]
    out = out * _BNS * gc_ref[...][None, :] + bec_ref[...][None, :]
    return jnp.maximum(out, 0.0)


def _layer0_kernel(xb_ref, q_ref, qn_ref, xn_ref, wxt_ref, bxt_ref, gxt_ref,
                   bexr_ref, wc_ref, bc_ref, gc_ref, bec_ref, out_ref):
    xb = xb_ref[0]
    q = q_ref[0]
    N = xb.shape[0]
    sels = _topk_dist(q, xb, qn_ref[0], xn_ref[0])
    iota = lax.broadcasted_iota(jnp.int32, sels[0].shape[:1] + (N,), 1)
    xg = [lax.dot_general((iota == s).astype(jnp.float32), xb,
                          (((1,), (0,)), ((), ())),
                          preferred_element_type=jnp.float32) for s in sels]
    s = _xform(jnp.concatenate([g[:, 0:1] for g in xg], axis=1),
               wxt_ref, bxt_ref, gxt_ref, bexr_ref)
    sb = _bf(s)
    xgb = [_bf(gj) for gj in xg]
    g = None
    for i in range(_K):
        si = sb[:, i * _K:(i + 1) * _K]
        terms = [si[:, j:j + 1] * xgb[j] for j in range(_K)]
        while len(terms) > 1:
            terms = [terms[k] + terms[k + 1] for k in range(0, len(terms), 2)]
        g = terms[0] if g is None else jnp.maximum(g, terms[0])
    out_ref[0] = _out_proj(g, wc_ref, bc_ref, gc_ref, bec_ref)


def _xconv_layer0(x, p, c_out, block_r=256):
    B, N, C = x.shape
    s2 = jnp.sum(x ** 2, axis=-1)
    full = lambda shape: pl.BlockSpec(shape, lambda b, r: (0,) * len(shape))
    return pl.pallas_call(
        _layer0_kernel,
        grid=(B, N // block_r),
        in_specs=[
            pl.BlockSpec((1, N, C), lambda b, r: (b, 0, 0)),
            pl.BlockSpec((1, block_r, C), lambda b, r: (b, r, 0)),
            pl.BlockSpec((1, block_r, 1), lambda b, r: (b, r, 0)),
            pl.BlockSpec((1, 1, N), lambda b, r: (b, 0, 0)),
            full((_K * _K, _K)), full((_K * _K,)), full((_K * _K,)),
            full((_K * _K,)),
            full((c_out, C)), full((c_out,)), full((c_out,)), full((c_out,)),
        ],
        out_specs=pl.BlockSpec((1, block_r, c_out), lambda b, r: (b, r, 0)),
        out_shape=jax.ShapeDtypeStruct((B, N, c_out), jnp.float32),
    )(x, x, s2[:, :, None], s2[:, None, :],
      p['Wxt'], p['bxt'], p['gxt'], p['betaxt'],
      p['Wc'], p['bc'], p['gc'], p['betac'])


def _knn_kernel(xb_ref, q_ref, qn_ref, xn_ref, idx_ref):
    xb = xb_ref[0]
    q = q_ref[0]
    N = xb.shape[0]
    b = pl.program_id(0)
    sels = _topk_dist(q, xb, qn_ref[0], xn_ref[0])
    idx_ref[0] = jnp.concatenate(sels, axis=1) + b * N


def _knn_stage(x, block_r=512):
    B, N, C = x.shape
    s2 = jnp.sum(x ** 2, axis=-1)
    return pl.pallas_call(
        _knn_kernel,
        grid=(B, N // block_r),
        in_specs=[
            pl.BlockSpec((1, N, C), lambda b, r: (b, 0, 0)),
            pl.BlockSpec((1, block_r, C), lambda b, r: (b, r, 0)),
            pl.BlockSpec((1, block_r, 1), lambda b, r: (b, r, 0)),
            pl.BlockSpec((1, 1, N), lambda b, r: (b, 0, 0)),
        ],
        out_specs=pl.BlockSpec((1, block_r, _K), lambda b, r: (b, r, 0)),
        out_shape=jax.ShapeDtypeStruct((B, N, _K), jnp.int32),
    )(x, x, s2[:, :, None], s2[:, None, :])


def _sc_gather(feat, idx3, c):
    nch = idx3.shape[1]
    total = _SC_NW * nch * _SC_CHUNK
    per_w = nch * _SC_CHUNK
    mesh = plsc.VectorSubcoreMesh(core_axis_name="c", subcore_axis_name="s")

    @functools.partial(
        pl.kernel,
        out_type=jax.ShapeDtypeStruct((total, c), jnp.float32),
        mesh=mesh,
        scratch_types=[
            pltpu.VMEM((nch, _SC_CHUNK), jnp.int32),
            pltpu.VMEM((_SC_CHUNK, c), jnp.float32),
            pltpu.SemaphoreType.DMA,
        ],
    )
    def gather_k(feat_hbm, idx_hbm, out_hbm, idx_v, rows_v, sem):
        wid = lax.axis_index("s") * _SC_NC + lax.axis_index("c")
        pltpu.sync_copy(idx_hbm.at[wid], idx_v)

        def body(t, carry):
            pltpu.async_copy(feat_hbm.at[idx_v.at[t]], rows_v, sem).wait()
            pltpu.sync_copy(
                rows_v,
                out_hbm.at[pl.ds(wid * per_w + t * _SC_CHUNK, _SC_CHUNK)])
            return carry

        lax.fori_loop(0, nch, body, 0)

    return gather_k(feat, idx3)


def _combine_kernel(g_ref, wxt_ref, bxt_ref, gxt_ref, bexr_ref,
                    wc_ref, bc_ref, gc_ref, bec_ref, out_ref):
    gav = [_bf(g_ref[:, j, :]) for j in range(_K)]
    xt = jnp.concatenate([g_ref[:, j, 0:1] for j in range(_K)], axis=1)
    s = _bf(_xform(xt, wxt_ref, bxt_ref, gxt_ref, bexr_ref))
    g = None
    for i in range(_K):
        si = s[:, i * _K:(i + 1) * _K]
        terms = [si[:, j:j + 1] * gav[j] for j in range(_K)]
        while len(terms) > 1:
            terms = [terms[k] + terms[k + 1] for k in range(0, len(terms), 2)]
        g = terms[0] if g is None else jnp.maximum(g, terms[0])
    out_ref[...] = _out_proj(g, wc_ref, bc_ref, gc_ref, bec_ref)


def _combine_stage(G, wc, p, c_out, block_r=512):
    M, _, C = G.shape
    full = lambda shape: pl.BlockSpec(shape, lambda r: (0,) * len(shape))
    return pl.pallas_call(
        _combine_kernel,
        grid=(M // block_r,),
        in_specs=[
            pl.BlockSpec((block_r, _K, C), lambda r: (r, 0, 0)),
            full((_K * _K, _K)), full((_K * _K,)), full((_K * _K,)),
            full((_K * _K,)),
            full((c_out, C)), full((c_out,)), full((c_out,)), full((c_out,)),
        ],
        out_specs=pl.BlockSpec((block_r, c_out), lambda r: (r, 0)),
        out_shape=jax.ShapeDtypeStruct((M, c_out), jnp.float32),
    )(G, p['Wxt'], p['bxt'], p['gxt'], p['betaxt'],
      wc, p['bc'], p['gc'], p['betac'])


def _head_kernel(h_ref, w1_ref, b1_ref, w2_ref, b2_ref, out_ref):
    h = h_ref[0]
    m = jnp.max(h, axis=0, keepdims=True)
    f = lax.dot_general(m, w1_ref[...], (((1,), (1,)), ((), ())),
                        preferred_element_type=jnp.float32)
    f = jnp.maximum(f + b1_ref[...][None, :], 0.0)
    o = lax.dot_general(f, w2_ref[...], (((1,), (1,)), ((), ())),
                        preferred_element_type=jnp.float32)
    out_ref[0, 0] = o[0] + b2_ref[...]


def _head(h, params):
    B, N, C = h.shape
    full = lambda shape: pl.BlockSpec(shape, lambda b: (0,) * len(shape))
    return pl.pallas_call(
        _head_kernel,
        grid=(B,),
        in_specs=[
            pl.BlockSpec((1, N, C), lambda b: (b, 0, 0)),
            full((256, 512)), full((256,)), full((40, 256)), full((40,)),
        ],
        out_specs=pl.BlockSpec((1, 1, 40), lambda b: (b, 0, 0)),
        out_shape=jax.ShapeDtypeStruct((B, 1, 40), jnp.float32),
    )(h, params['fc1_w'], params['fc1_b'], params['fc2_w'],
      params['fc2_b']).reshape(B, 40)


def _trunk(x, params):
    B, N, _ = x.shape
    h = _xconv_layer0(x, params['xconv0'], _CH[0][1])
    for i in range(1, len(_CH)):
        p = params['xconv%d' % i]
        C, c_out = _CH[i]
        idx = _knn_stage(h)
        idx3 = idx.reshape(_SC_NW, (B * N * _K) // (_SC_NW * _SC_CHUNK),
                           _SC_CHUNK)
        Cp = max(C, 128)
        feat = h.reshape(B * N, C)
        wc = p['Wc']
        if Cp != C:
            feat = jnp.pad(feat, ((0, 0), (0, Cp - C)))
            wc = jnp.pad(wc, ((0, 0), (0, Cp - C)))
        G = _sc_gather(feat, idx3, Cp)
        h = _combine_stage(G.reshape(B * N, _K, Cp), wc, p, c_out)
        h = h.reshape(B, N, c_out)
    return h


@jax.jit
def kernel(x, params):
    B = x.shape[0]
    h = jnp.concatenate([_trunk(x[:B // 2], params),
                         _trunk(x[B // 2:], params)], axis=0)
    return _head(h, params)


# speedup vs baseline: 1.1349x; 1.0267x over previous
import functools
import math

import jax
import jax.numpy as jnp
from jax import lax
from jax.experimental import pallas as pl
from jax.experimental.pallas import tpu as pltpu
from jax.experimental.pallas import tpu_sc as plsc

_EPS = 1e-5
_K = 8
_CH = [(3, 32), (32, 64), (64, 128), (128, 256), (256, 512)]
_BNS = float(1.0 / math.sqrt(1.0 + _EPS))

_SC_NC = 2
_SC_NS = 16
_SC_NW = _SC_NC * _SC_NS
_SC_CHUNK = 128


def _bf(a):
    return a.astype(jnp.bfloat16).astype(jnp.float32)


def _softmax_rows(a):
    m = jnp.max(a, axis=1, keepdims=True)
    e = jnp.exp(a - m)
    return e / jnp.sum(e, axis=1, keepdims=True)


def _topk_dist(q, xb, qn, xn):
    R = q.shape[0]
    N = xb.shape[0]
    d = -2.0 * lax.dot_general(q, xb, (((1,), (1,)), ((), ())),
                               preferred_element_type=jnp.float32)
    d = d + qn
    d = d + xn
    iota = lax.broadcasted_iota(jnp.int32, (R, N), 1)
    sels = []
    for _ in range(_K):
        m = jnp.min(d, axis=1, keepdims=True)
        sel = jnp.min(jnp.where(d == m, iota, N), axis=1, keepdims=True)
        sels.append(sel)
        d = jnp.where(iota == sel, jnp.float32(1e30), d)
    return sels


def _xform(xt, wxt_ref, bxt_ref, gxt_ref, bexr_ref):
    X = lax.dot_general(xt, wxt_ref[...], (((1,), (1,)), ((), ())),
                        preferred_element_type=jnp.float32)
    X = X + bxt_ref[...][None, :]
    X = X * _BNS * gxt_ref[...][None, :] + bexr_ref[...][None, :]
    return jnp.concatenate(
        [_softmax_rows(X[:, i * _K:(i + 1) * _K]) for i in range(_K)], axis=1)


def _out_proj(g, wc_ref, bc_ref, gc_ref, bec_ref):
    out = lax.dot_general(g, wc_ref[...], (((1,), (1,)), ((), ())),
                          preferred_element_type=jnp.float32)
    out = out + bc_ref[...][None, :]
    out = out * _BNS * gc_ref[...][None, :] + bec_ref[...][None, :]
    return jnp.maximum(out, 0.0)


def _layer0_kernel(xb_ref, q_ref, qn_ref, xn_ref, wxt_ref, bxt_ref, gxt_ref,
                   bexr_ref, wc_ref, bc_ref, gc_ref, bec_ref, out_ref):
    xb = xb_ref[0]
    q = q_ref[0]
    N = xb.shape[0]
    sels = _topk_dist(q, xb, qn_ref[0], xn_ref[0])
    iota = lax.broadcasted_iota(jnp.int32, sels[0].shape[:1] + (N,), 1)
    xg = [lax.dot_general((iota == s).astype(jnp.float32), xb,
                          (((1,), (0,)), ((), ())),
                          preferred_element_type=jnp.float32) for s in sels]
    s = _xform(jnp.concatenate([g[:, 0:1] for g in xg], axis=1),
               wxt_ref, bxt_ref, gxt_ref, bexr_ref)
    sb = _bf(s)
    xgb = [_bf(gj) for gj in xg]
    g = None
    for i in range(_K):
        si = sb[:, i * _K:(i + 1) * _K]
        terms = [si[:, j:j + 1] * xgb[j] for j in range(_K)]
        while len(terms) > 1:
            terms = [terms[k] + terms[k + 1] for k in range(0, len(terms), 2)]
        g = terms[0] if g is None else jnp.maximum(g, terms[0])
    out_ref[0] = _out_proj(g, wc_ref, bc_ref, gc_ref, bec_ref)


def _xconv_layer0(x, p, c_out, block_r=512):
    B, N, C = x.shape
    s2 = jnp.sum(x ** 2, axis=-1)
    full = lambda shape: pl.BlockSpec(shape, lambda b, r: (0,) * len(shape))
    return pl.pallas_call(
        _layer0_kernel,
        grid=(B, N // block_r),
        in_specs=[
            pl.BlockSpec((1, N, C), lambda b, r: (b, 0, 0)),
            pl.BlockSpec((1, block_r, C), lambda b, r: (b, r, 0)),
            pl.BlockSpec((1, block_r, 1), lambda b, r: (b, r, 0)),
            pl.BlockSpec((1, 1, N), lambda b, r: (b, 0, 0)),
            full((_K * _K, _K)), full((_K * _K,)), full((_K * _K,)),
            full((_K * _K,)),
            full((c_out, C)), full((c_out,)), full((c_out,)), full((c_out,)),
        ],
        out_specs=pl.BlockSpec((1, block_r, c_out), lambda b, r: (b, r, 0)),
        out_shape=jax.ShapeDtypeStruct((B, N, c_out), jnp.float32),
    )(x, x, s2[:, :, None], s2[:, None, :],
      p['Wxt'], p['bxt'], p['gxt'], p['betaxt'],
      p['Wc'], p['bc'], p['gc'], p['betac'])


def _knn_kernel(xb_ref, q_ref, qn_ref, xn_ref, idx_ref):
    xb = xb_ref[0]
    q = q_ref[0]
    N = xb.shape[0]
    b = pl.program_id(0)
    sels = _topk_dist(q, xb, qn_ref[0], xn_ref[0])
    idx_ref[0] = jnp.concatenate(sels, axis=1) + b * N


def _knn_stage(x, block_r=1024):
    B, N, C = x.shape
    s2 = jnp.sum(x ** 2, axis=-1)
    return pl.pallas_call(
        _knn_kernel,
        grid=(B, N // block_r),
        in_specs=[
            pl.BlockSpec((1, N, C), lambda b, r: (b, 0, 0)),
            pl.BlockSpec((1, block_r, C), lambda b, r: (b, r, 0)),
            pl.BlockSpec((1, block_r, 1), lambda b, r: (b, r, 0)),
            pl.BlockSpec((1, 1, N), lambda b, r: (b, 0, 0)),
        ],
        out_specs=pl.BlockSpec((1, block_r, _K), lambda b, r: (b, r, 0)),
        out_shape=jax.ShapeDtypeStruct((B, N, _K), jnp.int32),
    )(x, x, s2[:, :, None], s2[:, None, :])


def _sc_gather(feat, idx3, c):
    nch = idx3.shape[1]
    total = _SC_NW * nch * _SC_CHUNK
    per_w = nch * _SC_CHUNK
    mesh = plsc.VectorSubcoreMesh(core_axis_name="c", subcore_axis_name="s")

    @functools.partial(
        pl.kernel,
        out_type=jax.ShapeDtypeStruct((total, c), jnp.float32),
        mesh=mesh,
        scratch_types=[
            pltpu.VMEM((nch, _SC_CHUNK), jnp.int32),
            pltpu.VMEM((_SC_CHUNK, c), jnp.float32),
            pltpu.SemaphoreType.DMA,
        ],
    )
    def gather_k(feat_hbm, idx_hbm, out_hbm, idx_v, rows_v, sem):
        wid = lax.axis_index("s") * _SC_NC + lax.axis_index("c")
        pltpu.sync_copy(idx_hbm.at[wid], idx_v)

        def body(t, carry):
            pltpu.async_copy(feat_hbm.at[idx_v.at[t]], rows_v, sem).wait()
            pltpu.sync_copy(
                rows_v,
                out_hbm.at[pl.ds(wid * per_w + t * _SC_CHUNK, _SC_CHUNK)])
            return carry

        lax.fori_loop(0, nch, body, 0)

    return gather_k(feat, idx3)


def _combine_kernel(g_ref, wxt_ref, bxt_ref, gxt_ref, bexr_ref,
                    wc_ref, bc_ref, gc_ref, bec_ref, out_ref):
    gav = [_bf(g_ref[:, j, :]) for j in range(_K)]
    xt = jnp.concatenate([g_ref[:, j, 0:1] for j in range(_K)], axis=1)
    s = _bf(_xform(xt, wxt_ref, bxt_ref, gxt_ref, bexr_ref))
    g = None
    for i in range(_K):
        si = s[:, i * _K:(i + 1) * _K]
        terms = [si[:, j:j + 1] * gav[j] for j in range(_K)]
        while len(terms) > 1:
            terms = [terms[k] + terms[k + 1] for k in range(0, len(terms), 2)]
        g = terms[0] if g is None else jnp.maximum(g, terms[0])
    out_ref[...] = _out_proj(g, wc_ref, bc_ref, gc_ref, bec_ref)


def _combine_stage(G, wc, p, c_out, block_r=1024):
    M, _, C = G.shape
    full = lambda shape: pl.BlockSpec(shape, lambda r: (0,) * len(shape))
    return pl.pallas_call(
        _combine_kernel,
        grid=(M // block_r,),
        in_specs=[
            pl.BlockSpec((block_r, _K, C), lambda r: (r, 0, 0)),
            full((_K * _K, _K)), full((_K * _K,)), full((_K * _K,)),
            full((_K * _K,)),
            full((c_out, C)), full((c_out,)), full((c_out,)), full((c_out,)),
        ],
        out_specs=pl.BlockSpec((block_r, c_out), lambda r: (r, 0)),
        out_shape=jax.ShapeDtypeStruct((M, c_out), jnp.float32),
    )(G, p['Wxt'], p['bxt'], p['gxt'], p['betaxt'],
      wc, p['bc'], p['gc'], p['betac'])


def _head_kernel(h_ref, w1_ref, b1_ref, w2_ref, b2_ref, out_ref):
    h = h_ref[0]
    m = jnp.max(h, axis=0, keepdims=True)
    f = lax.dot_general(m, w1_ref[...], (((1,), (1,)), ((), ())),
                        preferred_element_type=jnp.float32)
    f = jnp.maximum(f + b1_ref[...][None, :], 0.0)
    o = lax.dot_general(f, w2_ref[...], (((1,), (1,)), ((), ())),
                        preferred_element_type=jnp.float32)
    out_ref[0, 0] = o[0] + b2_ref[...]


def _head(h, params):
    B, N, C = h.shape
    full = lambda shape: pl.BlockSpec(shape, lambda b: (0,) * len(shape))
    return pl.pallas_call(
        _head_kernel,
        grid=(B,),
        in_specs=[
            pl.BlockSpec((1, N, C), lambda b: (b, 0, 0)),
            full((256, 512)), full((256,)), full((40, 256)), full((40,)),
        ],
        out_specs=pl.BlockSpec((1, 1, 40), lambda b: (b, 0, 0)),
        out_shape=jax.ShapeDtypeStruct((B, 1, 40), jnp.float32),
    )(h, params['fc1_w'], params['fc1_b'], params['fc2_w'],
      params['fc2_b']).reshape(B, 40)


def _trunk(x, params):
    B, N, _ = x.shape
    h = _xconv_layer0(x, params['xconv0'], _CH[0][1])
    for i in range(1, len(_CH)):
        p = params['xconv%d' % i]
        C, c_out = _CH[i]
        idx = _knn_stage(h)
        idx3 = idx.reshape(_SC_NW, (B * N * _K) // (_SC_NW * _SC_CHUNK),
                           _SC_CHUNK)
        Cp = max(C, 128)
        feat = h.reshape(B * N, C)
        wc = p['Wc']
        if Cp != C:
            feat = jnp.pad(feat, ((0, 0), (0, Cp - C)))
            wc = jnp.pad(wc, ((0, 0), (0, Cp - C)))
        G = _sc_gather(feat, idx3, Cp)
        h = _combine_stage(G.reshape(B * N, _K, Cp), wc, p, c_out)
        h = h.reshape(B, N, c_out)
    return h


@jax.jit
def kernel(x, params):
    B = x.shape[0]
    h = jnp.concatenate([_trunk(x[:B // 2], params),
                         _trunk(x[B // 2:], params)], axis=0)
    return _head(h, params)


# speedup vs baseline: 1.1515x; 1.0146x over previous
import functools
import math

import jax
import jax.numpy as jnp
from jax import lax
from jax.experimental import pallas as pl
from jax.experimental.pallas import tpu as pltpu
from jax.experimental.pallas import tpu_sc as plsc

_EPS = 1e-5
_K = 8
_CH = [(3, 32), (32, 64), (64, 128), (128, 256), (256, 512)]
_BNS = float(1.0 / math.sqrt(1.0 + _EPS))

_SC_NC = 2
_SC_NS = 16
_SC_NW = _SC_NC * _SC_NS
_SC_CHUNK = 128


def _bf(a):
    return a.astype(jnp.bfloat16).astype(jnp.float32)


def _softmax_rows(a):
    m = jnp.max(a, axis=1, keepdims=True)
    e = jnp.exp(a - m)
    return e / jnp.sum(e, axis=1, keepdims=True)


def _topk_dist(q, xb, qn, xn):
    R = q.shape[0]
    N = xb.shape[0]
    d = -2.0 * lax.dot_general(q, xb, (((1,), (1,)), ((), ())),
                               preferred_element_type=jnp.float32)
    d = d + qn
    d = d + xn
    iota = lax.broadcasted_iota(jnp.int32, (R, N), 1)
    sels = []
    for _ in range(_K):
        m = jnp.min(d, axis=1, keepdims=True)
        sel = jnp.min(jnp.where(d == m, iota, N), axis=1, keepdims=True)
        sels.append(sel)
        d = jnp.where(iota == sel, jnp.float32(1e30), d)
    return sels


def _xform(xt, wxt_ref, bxt_ref, gxt_ref, bexr_ref):
    X = lax.dot_general(xt, wxt_ref[...], (((1,), (1,)), ((), ())),
                        preferred_element_type=jnp.float32)
    X = X + bxt_ref[...][None, :]
    X = X * _BNS * gxt_ref[...][None, :] + bexr_ref[...][None, :]
    return jnp.concatenate(
        [_softmax_rows(X[:, i * _K:(i + 1) * _K]) for i in range(_K)], axis=1)


def _out_proj(g, wc_ref, bc_ref, gc_ref, bec_ref):
    out = lax.dot_general(g, wc_ref[...], (((1,), (1,)), ((), ())),
                          preferred_element_type=jnp.float32)
    out = out + bc_ref[...][None, :]
    out = out * _BNS * gc_ref[...][None, :] + bec_ref[...][None, :]
    return jnp.maximum(out, 0.0)


def _layer0_kernel(xb_ref, q_ref, qn_ref, xn_ref, wxt_ref, bxt_ref, gxt_ref,
                   bexr_ref, wc_ref, bc_ref, gc_ref, bec_ref, out_ref):
    xb = xb_ref[0]
    q = q_ref[0]
    N = xb.shape[0]
    sels = _topk_dist(q, xb, qn_ref[0], xn_ref[0])
    iota = lax.broadcasted_iota(jnp.int32, sels[0].shape[:1] + (N,), 1)
    xg = [lax.dot_general((iota == s).astype(jnp.float32), xb,
                          (((1,), (0,)), ((), ())),
                          preferred_element_type=jnp.float32) for s in sels]
    s = _xform(jnp.concatenate([g[:, 0:1] for g in xg], axis=1),
               wxt_ref, bxt_ref, gxt_ref, bexr_ref)
    sb = _bf(s)
    xgb = [_bf(gj) for gj in xg]
    g = None
    for i in range(_K):
        si = sb[:, i * _K:(i + 1) * _K]
        terms = [si[:, j:j + 1] * xgb[j] for j in range(_K)]
        while len(terms) > 1:
            terms = [terms[k] + terms[k + 1] for k in range(0, len(terms), 2)]
        g = terms[0] if g is None else jnp.maximum(g, terms[0])
    out_ref[0] = _out_proj(g, wc_ref, bc_ref, gc_ref, bec_ref)


def _xconv_layer0(x, p, c_out, block_r=512):
    B, N, C = x.shape
    s2 = jnp.sum(x ** 2, axis=-1)
    full = lambda shape: pl.BlockSpec(shape, lambda b, r: (0,) * len(shape))
    return pl.pallas_call(
        _layer0_kernel,
        grid=(B, N // block_r),
        in_specs=[
            pl.BlockSpec((1, N, C), lambda b, r: (b, 0, 0)),
            pl.BlockSpec((1, block_r, C), lambda b, r: (b, r, 0)),
            pl.BlockSpec((1, block_r, 1), lambda b, r: (b, r, 0)),
            pl.BlockSpec((1, 1, N), lambda b, r: (b, 0, 0)),
            full((_K * _K, _K)), full((_K * _K,)), full((_K * _K,)),
            full((_K * _K,)),
            full((c_out, C)), full((c_out,)), full((c_out,)), full((c_out,)),
        ],
        out_specs=pl.BlockSpec((1, block_r, c_out), lambda b, r: (b, r, 0)),
        out_shape=jax.ShapeDtypeStruct((B, N, c_out), jnp.float32),
    )(x, x, s2[:, :, None], s2[:, None, :],
      p['Wxt'], p['bxt'], p['gxt'], p['betaxt'],
      p['Wc'], p['bc'], p['gc'], p['betac'])


def _knn_kernel(xb_ref, q_ref, qn_ref, xn_ref, idx_ref):
    xb = xb_ref[0]
    q = q_ref[0]
    N = xb.shape[0]
    b = pl.program_id(0)
    sels = _topk_dist(q, xb, qn_ref[0], xn_ref[0])
    idx_ref[0] = jnp.concatenate(sels, axis=1) + b * N


def _knn_stage(x, block_r=1024):
    B, N, C = x.shape
    s2 = jnp.sum(x ** 2, axis=-1)
    return pl.pallas_call(
        _knn_kernel,
        grid=(B, N // block_r),
        in_specs=[
            pl.BlockSpec((1, N, C), lambda b, r: (b, 0, 0)),
            pl.BlockSpec((1, block_r, C), lambda b, r: (b, r, 0)),
            pl.BlockSpec((1, block_r, 1), lambda b, r: (b, r, 0)),
            pl.BlockSpec((1, 1, N), lambda b, r: (b, 0, 0)),
        ],
        out_specs=pl.BlockSpec((1, block_r, _K), lambda b, r: (b, r, 0)),
        out_shape=jax.ShapeDtypeStruct((B, N, _K), jnp.int32),
    )(x, x, s2[:, :, None], s2[:, None, :])


def _sc_gather(feat, idx3, c):
    nch = idx3.shape[1]
    total = _SC_NW * nch * _SC_CHUNK
    per_w = nch * _SC_CHUNK
    mesh = plsc.VectorSubcoreMesh(core_axis_name="c", subcore_axis_name="s")

    @functools.partial(
        pl.kernel,
        out_type=jax.ShapeDtypeStruct((total, c), jnp.float32),
        mesh=mesh,
        scratch_types=[
            pltpu.VMEM((nch, _SC_CHUNK), jnp.int32),
            pltpu.VMEM((_SC_CHUNK, c), jnp.float32),
            pltpu.SemaphoreType.DMA,
        ],
    )
    def gather_k(feat_hbm, idx_hbm, out_hbm, idx_v, rows_v, sem):
        wid = lax.axis_index("s") * _SC_NC + lax.axis_index("c")
        pltpu.sync_copy(idx_hbm.at[wid], idx_v)

        def body(t, carry):
            pltpu.async_copy(feat_hbm.at[idx_v.at[t]], rows_v, sem).wait()
            pltpu.sync_copy(
                rows_v,
                out_hbm.at[pl.ds(wid * per_w + t * _SC_CHUNK, _SC_CHUNK)])
            return carry

        lax.fori_loop(0, nch, body, 0)

    return gather_k(feat, idx3)


def _combine_kernel(g_ref, wxt_ref, bxt_ref, gxt_ref, bexr_ref,
                    wc_ref, bc_ref, gc_ref, bec_ref, out_ref):
    gav = [_bf(g_ref[:, j, :]) for j in range(_K)]
    xt = jnp.concatenate([g_ref[:, j, 0:1] for j in range(_K)], axis=1)
    s = _bf(_xform(xt, wxt_ref, bxt_ref, gxt_ref, bexr_ref))
    g = None
    for i in range(_K):
        si = s[:, i * _K:(i + 1) * _K]
        terms = [si[:, j:j + 1] * gav[j] for j in range(_K)]
        while len(terms) > 1:
            terms = [terms[k] + terms[k + 1] for k in range(0, len(terms), 2)]
        g = terms[0] if g is None else jnp.maximum(g, terms[0])
    out_ref[...] = _out_proj(g, wc_ref, bc_ref, gc_ref, bec_ref)


def _combine_max_kernel(g_ref, wxt_ref, bxt_ref, gxt_ref, bexr_ref,
                        wc_ref, bc_ref, gc_ref, bec_ref, out_ref):
    gav = [_bf(g_ref[:, j, :]) for j in range(_K)]
    xt = jnp.concatenate([g_ref[:, j, 0:1] for j in range(_K)], axis=1)
    s = _bf(_xform(xt, wxt_ref, bxt_ref, gxt_ref, bexr_ref))
    g = None
    for i in range(_K):
        si = s[:, i * _K:(i + 1) * _K]
        terms = [si[:, j:j + 1] * gav[j] for j in range(_K)]
        while len(terms) > 1:
            terms = [terms[k] + terms[k + 1] for k in range(0, len(terms), 2)]
        g = terms[0] if g is None else jnp.maximum(g, terms[0])
    out = _out_proj(g, wc_ref, bc_ref, gc_ref, bec_ref)
    out_ref[0] = jnp.max(out, axis=0, keepdims=True)


def _combine_stage(G, wc, p, c_out, block_r=1024, final_max=False):
    M, _, C = G.shape
    full = lambda shape: pl.BlockSpec(shape, lambda r: (0,) * len(shape))
    in_specs = [
        pl.BlockSpec((block_r, _K, C), lambda r: (r, 0, 0)),
        full((_K * _K, _K)), full((_K * _K,)), full((_K * _K,)),
        full((_K * _K,)),
        full((c_out, C)), full((c_out,)), full((c_out,)), full((c_out,)),
    ]
    args = (G, p['Wxt'], p['bxt'], p['gxt'], p['betaxt'],
            wc, p['bc'], p['gc'], p['betac'])
    if final_max:
        return pl.pallas_call(
            _combine_max_kernel,
            grid=(M // block_r,),
            in_specs=in_specs,
            out_specs=pl.BlockSpec((1, 1, c_out), lambda r: (r, 0, 0)),
            out_shape=jax.ShapeDtypeStruct((M // block_r, 1, c_out),
                                           jnp.float32),
        )(*args)
    return pl.pallas_call(
        _combine_kernel,
        grid=(M // block_r,),
        in_specs=in_specs,
        out_specs=pl.BlockSpec((block_r, c_out), lambda r: (r, 0)),
        out_shape=jax.ShapeDtypeStruct((M, c_out), jnp.float32),
    )(*args)


def _head_kernel(h_ref, w1_ref, b1_ref, w2_ref, b2_ref, out_ref):
    m = h_ref[0]
    f = lax.dot_general(m, w1_ref[...], (((1,), (1,)), ((), ())),
                        preferred_element_type=jnp.float32)
    f = jnp.maximum(f + b1_ref[...][None, :], 0.0)
    o = lax.dot_general(f, w2_ref[...], (((1,), (1,)), ((), ())),
                        preferred_element_type=jnp.float32)
    out_ref[0, 0] = o[0] + b2_ref[...]


def _head(h, params):
    B, _, C = h.shape
    full = lambda shape: pl.BlockSpec(shape, lambda b: (0,) * len(shape))
    return pl.pallas_call(
        _head_kernel,
        grid=(B,),
        in_specs=[
            pl.BlockSpec((1, 1, C), lambda b: (b, 0, 0)),
            full((256, 512)), full((256,)), full((40, 256)), full((40,)),
        ],
        out_specs=pl.BlockSpec((1, 1, 40), lambda b: (b, 0, 0)),
        out_shape=jax.ShapeDtypeStruct((B, 1, 40), jnp.float32),
    )(h, params['fc1_w'], params['fc1_b'], params['fc2_w'],
      params['fc2_b']).reshape(B, 40)


def _trunk(x, params):
    B, N, _ = x.shape
    h = _xconv_layer0(x, params['xconv0'], _CH[0][1])
    for i in range(1, len(_CH)):
        p = params['xconv%d' % i]
        C, c_out = _CH[i]
        idx = _knn_stage(h)
        idx3 = idx.reshape(_SC_NW, (B * N * _K) // (_SC_NW * _SC_CHUNK),
                           _SC_CHUNK)
        Cp = max(C, 128)
        feat = h.reshape(B * N, C)
        wc = p['Wc']
        if Cp != C:
            feat = jnp.pad(feat, ((0, 0), (0, Cp - C)))
            wc = jnp.pad(wc, ((0, 0), (0, Cp - C)))
        G = _sc_gather(feat, idx3, Cp)
        final = i == len(_CH) - 1
        h = _combine_stage(G.reshape(B * N, _K, Cp), wc, p, c_out,
                           block_r=N, final_max=final)
        if final:
            return h
        h = h.reshape(B, N, c_out)
    return h


@jax.jit
def kernel(x, params):
    B = x.shape[0]
    h = jnp.concatenate([_trunk(x[:B // 2], params),
                         _trunk(x[B // 2:], params)], axis=0)
    return _head(h, params)


# speedup vs baseline: 1.1853x; 1.0294x over previous
import functools
import math

import jax
import jax.numpy as jnp
from jax import lax
from jax.experimental import pallas as pl
from jax.experimental.pallas import tpu as pltpu
from jax.experimental.pallas import tpu_sc as plsc

_EPS = 1e-5
_K = 8
_CH = [(3, 32), (32, 64), (64, 128), (128, 256), (256, 512)]
_BNS = float(1.0 / math.sqrt(1.0 + _EPS))

_SC_NC = 2
_SC_NS = 16
_SC_NW = _SC_NC * _SC_NS
_SC_CHUNK = 128


def _bf(a):
    return a.astype(jnp.bfloat16).astype(jnp.float32)


def _softmax_rows(a):
    m = jnp.max(a, axis=1, keepdims=True)
    e = jnp.exp(a - m)
    return e / jnp.sum(e, axis=1, keepdims=True)


def _topk_dist(q, xb, qn, xn):
    R = q.shape[0]
    N = xb.shape[0]
    d = -2.0 * lax.dot_general(q, xb, (((1,), (1,)), ((), ())),
                               preferred_element_type=jnp.float32)
    d = d + qn
    d = d + xn
    iota = lax.broadcasted_iota(jnp.int32, (R, N), 1)
    sels = []
    for _ in range(_K):
        m = jnp.min(d, axis=1, keepdims=True)
        sel = jnp.min(jnp.where(d == m, iota, N), axis=1, keepdims=True)
        sels.append(sel)
        d = jnp.where(iota == sel, jnp.float32(1e30), d)
    return sels


def _xform(xt, wxt_ref, bxt_ref, gxt_ref, bexr_ref):
    X = lax.dot_general(xt, wxt_ref[...], (((1,), (1,)), ((), ())),
                        preferred_element_type=jnp.float32)
    X = X + bxt_ref[...][None, :]
    X = X * _BNS * gxt_ref[...][None, :] + bexr_ref[...][None, :]
    return jnp.concatenate(
        [_softmax_rows(X[:, i * _K:(i + 1) * _K]) for i in range(_K)], axis=1)


def _out_proj(g, wc_ref, bc_ref, gc_ref, bec_ref):
    out = lax.dot_general(g, wc_ref[...], (((1,), (1,)), ((), ())),
                          preferred_element_type=jnp.float32)
    out = out + bc_ref[...][None, :]
    out = out * _BNS * gc_ref[...][None, :] + bec_ref[...][None, :]
    return jnp.maximum(out, 0.0)


def _layer0_kernel(xb_ref, q_ref, qn_ref, xn_ref, wxt_ref, bxt_ref, gxt_ref,
                   bexr_ref, wc_ref, bc_ref, gc_ref, bec_ref, out_ref):
    xb = xb_ref[0]
    q = q_ref[0]
    N = xb.shape[0]
    sels = _topk_dist(q, xb, qn_ref[0], xn_ref[0])
    iota = lax.broadcasted_iota(jnp.int32, sels[0].shape[:1] + (N,), 1)
    xg = [lax.dot_general((iota == s).astype(jnp.float32), xb,
                          (((1,), (0,)), ((), ())),
                          preferred_element_type=jnp.float32) for s in sels]
    s = _xform(jnp.concatenate([g[:, 0:1] for g in xg], axis=1),
               wxt_ref, bxt_ref, gxt_ref, bexr_ref)
    sb = _bf(s)
    xgb = [_bf(gj) for gj in xg]
    g = None
    for i in range(_K):
        si = sb[:, i * _K:(i + 1) * _K]
        terms = [si[:, j:j + 1] * xgb[j] for j in range(_K)]
        while len(terms) > 1:
            terms = [terms[k] + terms[k + 1] for k in range(0, len(terms), 2)]
        g = terms[0] if g is None else jnp.maximum(g, terms[0])
    out_ref[0] = _out_proj(g, wc_ref, bc_ref, gc_ref, bec_ref)


def _xconv_layer0(x, p, c_out, block_r=1024):
    B, N, C = x.shape
    s2 = jnp.sum(x ** 2, axis=-1)
    full = lambda shape: pl.BlockSpec(shape, lambda b, r: (0,) * len(shape))
    return pl.pallas_call(
        _layer0_kernel,
        grid=(B, N // block_r),
        in_specs=[
            pl.BlockSpec((1, N, C), lambda b, r: (b, 0, 0)),
            pl.BlockSpec((1, block_r, C), lambda b, r: (b, r, 0)),
            pl.BlockSpec((1, block_r, 1), lambda b, r: (b, r, 0)),
            pl.BlockSpec((1, 1, N), lambda b, r: (b, 0, 0)),
            full((_K * _K, _K)), full((_K * _K,)), full((_K * _K,)),
            full((_K * _K,)),
            full((c_out, C)), full((c_out,)), full((c_out,)), full((c_out,)),
        ],
        out_specs=pl.BlockSpec((1, block_r, c_out), lambda b, r: (b, r, 0)),
        out_shape=jax.ShapeDtypeStruct((B, N, c_out), jnp.float32),
    )(x, x, s2[:, :, None], s2[:, None, :],
      p['Wxt'], p['bxt'], p['gxt'], p['betaxt'],
      p['Wc'], p['bc'], p['gc'], p['betac'])


def _knn_kernel(xb_ref, q_ref, qn_ref, xn_ref, idx_ref):
    xb = xb_ref[0]
    q = q_ref[0]
    N = xb.shape[0]
    b = pl.program_id(0)
    sels = _topk_dist(q, xb, qn_ref[0], xn_ref[0])
    idx_ref[0] = jnp.concatenate(sels, axis=1) + b * N


def _knn_stage(x, block_r=1024):
    B, N, C = x.shape
    s2 = jnp.sum(x ** 2, axis=-1)
    return pl.pallas_call(
        _knn_kernel,
        grid=(B, N // block_r),
        in_specs=[
            pl.BlockSpec((1, N, C), lambda b, r: (b, 0, 0)),
            pl.BlockSpec((1, block_r, C), lambda b, r: (b, r, 0)),
            pl.BlockSpec((1, block_r, 1), lambda b, r: (b, r, 0)),
            pl.BlockSpec((1, 1, N), lambda b, r: (b, 0, 0)),
        ],
        out_specs=pl.BlockSpec((1, block_r, _K), lambda b, r: (b, r, 0)),
        out_shape=jax.ShapeDtypeStruct((B, N, _K), jnp.int32),
    )(x, x, s2[:, :, None], s2[:, None, :])


def _sc_gather(feat, idx3, c):
    nch = idx3.shape[1]
    total = _SC_NW * nch * _SC_CHUNK
    per_w = nch * _SC_CHUNK
    mesh = plsc.VectorSubcoreMesh(core_axis_name="c", subcore_axis_name="s")

    @functools.partial(
        pl.kernel,
        out_type=jax.ShapeDtypeStruct((total, c), jnp.float32),
        mesh=mesh,
        scratch_types=[
            pltpu.VMEM((nch, _SC_CHUNK), jnp.int32),
            pltpu.VMEM((_SC_CHUNK, c), jnp.float32),
            pltpu.SemaphoreType.DMA,
        ],
    )
    def gather_k(feat_hbm, idx_hbm, out_hbm, idx_v, rows_v, sem):
        wid = lax.axis_index("s") * _SC_NC + lax.axis_index("c")
        pltpu.sync_copy(idx_hbm.at[wid], idx_v)

        def body(t, carry):
            pltpu.async_copy(feat_hbm.at[idx_v.at[t]], rows_v, sem).wait()
            pltpu.sync_copy(
                rows_v,
                out_hbm.at[pl.ds(wid * per_w + t * _SC_CHUNK, _SC_CHUNK)])
            return carry

        lax.fori_loop(0, nch, body, 0)

    return gather_k(feat, idx3)


def _combine_kernel(g_ref, wxt_ref, bxt_ref, gxt_ref, bexr_ref,
                    wc_ref, bc_ref, gc_ref, bec_ref, out_ref):
    gav = [_bf(g_ref[:, j, :]) for j in range(_K)]
    xt = jnp.concatenate([g_ref[:, j, 0:1] for j in range(_K)], axis=1)
    s = _bf(_xform(xt, wxt_ref, bxt_ref, gxt_ref, bexr_ref))
    g = None
    for i in range(_K):
        si = s[:, i * _K:(i + 1) * _K]
        terms = [si[:, j:j + 1] * gav[j] for j in range(_K)]
        while len(terms) > 1:
            terms = [terms[k] + terms[k + 1] for k in range(0, len(terms), 2)]
        g = terms[0] if g is None else jnp.maximum(g, terms[0])
    out_ref[...] = _out_proj(g, wc_ref, bc_ref, gc_ref, bec_ref)


def _combine_max_kernel(g_ref, wxt_ref, bxt_ref, gxt_ref, bexr_ref,
                        wc_ref, bc_ref, gc_ref, bec_ref, out_ref):
    gav = [_bf(g_ref[:, j, :]) for j in range(_K)]
    xt = jnp.concatenate([g_ref[:, j, 0:1] for j in range(_K)], axis=1)
    s = _bf(_xform(xt, wxt_ref, bxt_ref, gxt_ref, bexr_ref))
    g = None
    for i in range(_K):
        si = s[:, i * _K:(i + 1) * _K]
        terms = [si[:, j:j + 1] * gav[j] for j in range(_K)]
        while len(terms) > 1:
            terms = [terms[k] + terms[k + 1] for k in range(0, len(terms), 2)]
        g = terms[0] if g is None else jnp.maximum(g, terms[0])
    out = _out_proj(g, wc_ref, bc_ref, gc_ref, bec_ref)
    out_ref[0] = jnp.max(out, axis=0, keepdims=True)


def _combine_stage(G, wc, p, c_out, block_r=1024, final_max=False):
    M, _, C = G.shape
    full = lambda shape: pl.BlockSpec(shape, lambda r: (0,) * len(shape))
    in_specs = [
        pl.BlockSpec((block_r, _K, C), lambda r: (r, 0, 0)),
        full((_K * _K, _K)), full((_K * _K,)), full((_K * _K,)),
        full((_K * _K,)),
        full((c_out, C)), full((c_out,)), full((c_out,)), full((c_out,)),
    ]
    args = (G, p['Wxt'], p['bxt'], p['gxt'], p['betaxt'],
            wc, p['bc'], p['gc'], p['betac'])
    if final_max:
        return pl.pallas_call(
            _combine_max_kernel,
            grid=(M // block_r,),
            in_specs=in_specs,
            out_specs=pl.BlockSpec((1, 1, c_out), lambda r: (r, 0, 0)),
            out_shape=jax.ShapeDtypeStruct((M // block_r, 1, c_out),
                                           jnp.float32),
        )(*args)
    return pl.pallas_call(
        _combine_kernel,
        grid=(M // block_r,),
        in_specs=in_specs,
        out_specs=pl.BlockSpec((block_r, c_out), lambda r: (r, 0)),
        out_shape=jax.ShapeDtypeStruct((M, c_out), jnp.float32),
    )(*args)


def _head_kernel(h_ref, w1_ref, b1_ref, w2_ref, b2_ref, out_ref):
    m = h_ref[0]
    f = lax.dot_general(m, w1_ref[...], (((1,), (1,)), ((), ())),
                        preferred_element_type=jnp.float32)
    f = jnp.maximum(f + b1_ref[...][None, :], 0.0)
    o = lax.dot_general(f, w2_ref[...], (((1,), (1,)), ((), ())),
                        preferred_element_type=jnp.float32)
    out_ref[0, 0] = o[0] + b2_ref[...]


def _head(h, params):
    B, _, C = h.shape
    full = lambda shape: pl.BlockSpec(shape, lambda b: (0,) * len(shape))
    return pl.pallas_call(
        _head_kernel,
        grid=(B,),
        in_specs=[
            pl.BlockSpec((1, 1, C), lambda b: (b, 0, 0)),
            full((256, 512)), full((256,)), full((40, 256)), full((40,)),
        ],
        out_specs=pl.BlockSpec((1, 1, 40), lambda b: (b, 0, 0)),
        out_shape=jax.ShapeDtypeStruct((B, 1, 40), jnp.float32),
    )(h, params['fc1_w'], params['fc1_b'], params['fc2_w'],
      params['fc2_b']).reshape(B, 40)


def _trunk(x, params):
    B, N, _ = x.shape
    h = _xconv_layer0(x, params['xconv0'], _CH[0][1])
    for i in range(1, len(_CH)):
        p = params['xconv%d' % i]
        C, c_out = _CH[i]
        idx = _knn_stage(h)
        idx3 = idx.reshape(_SC_NW, (B * N * _K) // (_SC_NW * _SC_CHUNK),
                           _SC_CHUNK)
        Cp = max(C, 128)
        feat = h.reshape(B * N, C)
        wc = p['Wc']
        if Cp != C:
            feat = jnp.pad(feat, ((0, 0), (0, Cp - C)))
            wc = jnp.pad(wc, ((0, 0), (0, Cp - C)))
        G = _sc_gather(feat, idx3, Cp)
        final = i == len(_CH) - 1
        h = _combine_stage(G.reshape(B * N, _K, Cp), wc, p, c_out,
                           block_r=N, final_max=final)
        if final:
            return h
        h = h.reshape(B, N, c_out)
    return h


@jax.jit
def kernel(x, params):
    B = x.shape[0]
    h = jnp.concatenate([_trunk(x[:B // 2], params),
                         _trunk(x[B // 2:], params)], axis=0)
    return _head(h, params)
